# baseline (speedup 1.0000x reference)
import jax
import jax.numpy as jnp
from jax.experimental import pallas as pl
from jax.experimental.pallas import tpu as pltpu

_B, _C, _N = 4, 32, 4096
_TM = 512
_J = _N // _TM
_K = 40


def _chamfer_body(inp_ref, tgt_ref, maskx_ref, masky_ref, out_ref,
                  acc_ref, a_ref):
    b = pl.program_id(0)

    xm = inp_ref[0] * maskx_ref[0]
    xx = jnp.sum(xm * xm, axis=0)
    a_ref[...] = jnp.concatenate(
        [xm, xx[None, :], jnp.ones((1, _N), jnp.float32),
         jnp.zeros((_K - _C - 2, _N), jnp.float32)],
        axis=0).astype(jnp.bfloat16)

    @pl.when(b == 0)
    def _init():
        acc_ref[0] = 0.0

    a = a_ref[...]
    dims = (((0,), (0,)), ((), ()))
    dist1 = None
    for j in range(_J):
        ym = tgt_ref[0, :, j * _TM:(j + 1) * _TM] \
            * masky_ref[0, :, j * _TM:(j + 1) * _TM]
        yy = jnp.sum(ym * ym, axis=0)
        bm = jnp.concatenate(
            [-2.0 * ym, jnp.ones((1, _TM), jnp.float32), yy[None, :],
             jnp.zeros((_K - _C - 2, _TM), jnp.float32)],
            axis=0).astype(jnp.bfloat16)
        dt = jax.lax.dot_general(
            bm, a, dims, preferred_element_type=jnp.float32)
        colmin = jnp.min(dt, axis=1)
        tilemin = jnp.min(dt, axis=0)
        acc_ref[0] += jnp.sum(jnp.maximum(colmin, 0.0))
        dist1 = tilemin if dist1 is None else jnp.minimum(dist1, tilemin)

    acc_ref[0] += jnp.sum(jnp.maximum(dist1, 0.0))

    @pl.when(b == _B - 1)
    def _emit():
        out_ref[0, 0] = acc_ref[0] * (1.0 / (_B * _N))


def kernel(inp, tgt, mask):
    mask3 = mask.reshape(_B, 1, _N)
    out = pl.pallas_call(
        _chamfer_body,
        grid=(_B,),
        in_specs=[
            pl.BlockSpec((1, _C, _N), lambda b: (b, 0, 0)),
            pl.BlockSpec((1, _C, _N), lambda b: (b, 0, 0)),
            pl.BlockSpec((1, 1, _N), lambda b: (b, 0, 0)),
            pl.BlockSpec((1, 1, _N), lambda b: (b, 0, 0)),
        ],
        out_specs=pl.BlockSpec(
            (1, 1), lambda b: (0, 0), memory_space=pltpu.SMEM),
        out_shape=jax.ShapeDtypeStruct((1, 1), jnp.float32),
        scratch_shapes=[
            pltpu.SMEM((1,), jnp.float32),
            pltpu.VMEM((_K, _N), jnp.bfloat16),
        ],
        compiler_params=pltpu.CompilerParams(
            dimension_semantics=("arbitrary",),
            vmem_limit_bytes=63 * 1024 * 1024),
    )(inp, tgt, mask3, mask3)
    return out[0, 0]


# speedup vs baseline: 1.0091x; 1.0091x over previous
import jax
import jax.numpy as jnp
from jax.experimental import pallas as pl
from jax.experimental.pallas import tpu as pltpu

_B, _C, _N = 4, 32, 4096
_TM = 2048
_J = _N // _TM
_K = 40


def _chamfer_body(inp_ref, tgt_ref, maskx_ref, masky_ref, out_ref,
                  acc_ref, a_ref):
    b = pl.program_id(0)

    xm = inp_ref[0] * maskx_ref[0]
    xx = jnp.sum(xm * xm, axis=0)
    a_ref[...] = jnp.concatenate(
        [xm, xx[None, :], jnp.ones((1, _N), jnp.float32),
         jnp.zeros((_K - _C - 2, _N), jnp.float32)],
        axis=0).astype(jnp.bfloat16)

    @pl.when(b == 0)
    def _init():
        acc_ref[0] = 0.0

    a = a_ref[...]
    dims = (((0,), (0,)), ((), ()))
    dist1 = None
    for j in range(_J):
        ym = tgt_ref[0, :, j * _TM:(j + 1) * _TM] \
            * masky_ref[0, :, j * _TM:(j + 1) * _TM]
        yy = jnp.sum(ym * ym, axis=0)
        bm = jnp.concatenate(
            [-2.0 * ym, jnp.ones((1, _TM), jnp.float32), yy[None, :],
             jnp.zeros((_K - _C - 2, _TM), jnp.float32)],
            axis=0).astype(jnp.bfloat16)
        dt = jax.lax.dot_general(
            bm, a, dims, preferred_element_type=jnp.float32)
        colmin = jnp.min(dt, axis=1)
        tilemin = jnp.min(dt, axis=0)
        acc_ref[0] += jnp.sum(jnp.maximum(colmin, 0.0))
        dist1 = tilemin if dist1 is None else jnp.minimum(dist1, tilemin)

    acc_ref[0] += jnp.sum(jnp.maximum(dist1, 0.0))

    @pl.when(b == _B - 1)
    def _emit():
        out_ref[0, 0] = acc_ref[0] * (1.0 / (_B * _N))


def kernel(inp, tgt, mask):
    mask3 = mask.reshape(_B, 1, _N)
    out = pl.pallas_call(
        _chamfer_body,
        grid=(_B,),
        in_specs=[
            pl.BlockSpec((1, _C, _N), lambda b: (b, 0, 0)),
            pl.BlockSpec((1, _C, _N), lambda b: (b, 0, 0)),
            pl.BlockSpec((1, 1, _N), lambda b: (b, 0, 0)),
            pl.BlockSpec((1, 1, _N), lambda b: (b, 0, 0)),
        ],
        out_specs=pl.BlockSpec(
            (1, 1), lambda b: (0, 0), memory_space=pltpu.SMEM),
        out_shape=jax.ShapeDtypeStruct((1, 1), jnp.float32),
        scratch_shapes=[
            pltpu.SMEM((1,), jnp.float32),
            pltpu.VMEM((_K, _N), jnp.bfloat16),
        ],
        compiler_params=pltpu.CompilerParams(
            dimension_semantics=("arbitrary",),
            vmem_limit_bytes=63 * 1024 * 1024),
    )(inp, tgt, mask3, mask3)
    return out[0, 0]
